# baseline (speedup 1.0000x reference)
import functools

import jax
import jax.numpy as jnp
from jax import lax
from jax.experimental import pallas as pl
from jax.experimental.pallas import tpu as pltpu
from jax.experimental.pallas import tpu_sc as plsc

N, E, DI, DO = 4096, 32768, 128, 128
EPS = 1e-06
EB = 1024
CH = 128
AW = 8
DZ = DO + AW
CK = 4
ECK = E // CK


def _make_gather(nc, ns):
    nw = nc * ns
    epw = ECK // nw
    chw = epw // CH

    def body(x_hbm, src_hbm, tgt_hbm, xs_hbm, xt_hbm,
             sidx, tidx, buf_a, buf_b, sem_a, sem_b):
        wid = lax.axis_index("s") * nc + lax.axis_index("c")
        base = wid * epw
        pltpu.sync_copy(src_hbm.at[pl.ds(wid * chw, chw)], sidx)
        pltpu.sync_copy(tgt_hbm.at[pl.ds(wid * chw, chw)], tidx)
        for k in range(chw):
            ca = pltpu.async_copy(x_hbm.at[sidx.at[k]], buf_a, sem_a)
            cb = pltpu.async_copy(x_hbm.at[tidx.at[k]], buf_b, sem_b)
            ca.wait()
            pltpu.sync_copy(buf_a, xs_hbm.at[pl.ds(base + k * CH, CH)])
            cb.wait()
            pltpu.sync_copy(buf_b, xt_hbm.at[pl.ds(base + k * CH, CH)])

    return pl.kernel(
        body,
        out_type=[
            jax.ShapeDtypeStruct((ECK, DI), jnp.float32),
            jax.ShapeDtypeStruct((ECK, DI), jnp.float32),
        ],
        mesh=plsc.VectorSubcoreMesh(core_axis_name="c", subcore_axis_name="s"),
        scratch_types=[
            pltpu.VMEM((chw, CH), jnp.int32),
            pltpu.VMEM((chw, CH), jnp.int32),
            pltpu.VMEM((CH, DI), jnp.float32),
            pltpu.VMEM((CH, DI), jnp.float32),
            pltpu.SemaphoreType.DMA,
            pltpu.SemaphoreType.DMA,
        ],
    )


def _agg_body(mtgt_ref, xs_ref, xt_ref, w1_ref, w2_ref, b_ref, acc_out_ref,
              acc_ref):
    i = pl.program_id(0)

    @pl.when(i == 0)
    def _init():
        acc_ref[...] = jnp.zeros_like(acc_ref)

    f = (jnp.dot(xs_ref[...], w1_ref[...], preferred_element_type=jnp.float32)
         + jnp.dot(xt_ref[...], w2_ref[...], preferred_element_type=jnp.float32)
         + b_ref[...])
    y = jnp.maximum(f[:, :DO], 0.0)
    a = f[:, DO:DO + 1]
    z = jnp.concatenate([y * a, f[:, DO:]], axis=1)
    acc_ref[...] += jnp.dot(mtgt_ref[...], z, preferred_element_type=jnp.float32)

    @pl.when(i == pl.num_programs(0) - 1)
    def _fini():
        acc_out_ref[...] = acc_ref[...]


def _combine_body(a0_ref, a1_ref, a2_ref, a3_ref, o_ref):
    s = a0_ref[...] + a1_ref[...] + a2_ref[...] + a3_ref[...]
    o_ref[...] = s[:, :DO] / (s[:, DO:DO + 1] + EPS)


@functools.partial(jax.jit, static_argnames=("nc", "ns"))
def _run(x, src2d, tgt2d, Mtgt, W1a, W2a, bvec, nc, ns):
    gather = _make_gather(nc, ns)
    rows = ECK // CH
    nblk = ECK // EB
    accs = []
    for c in range(CK):
        xs, xt = gather(x,
                        lax.slice(src2d, (c * rows, 0), ((c + 1) * rows, CH)),
                        lax.slice(tgt2d, (c * rows, 0), ((c + 1) * rows, CH)))
        acc = pl.pallas_call(
            _agg_body,
            grid=(nblk,),
            in_specs=[
                pl.BlockSpec((N, EB), lambda i, c=c: (0, c * nblk + i)),
                pl.BlockSpec((EB, DI), lambda i: (i, 0)),
                pl.BlockSpec((EB, DI), lambda i: (i, 0)),
                pl.BlockSpec((DI, DZ), lambda i: (0, 0)),
                pl.BlockSpec((DI, DZ), lambda i: (0, 0)),
                pl.BlockSpec((1, DZ), lambda i: (0, 0)),
            ],
            out_specs=pl.BlockSpec((N, DZ), lambda i: (0, 0)),
            out_shape=jax.ShapeDtypeStruct((N, DZ), jnp.float32),
            scratch_shapes=[pltpu.VMEM((N, DZ), jnp.float32)],
        )(Mtgt, xs, xt, W1a, W2a, bvec)
        accs.append(acc)

    o = pl.pallas_call(
        _combine_body,
        out_shape=jax.ShapeDtypeStruct((N, DO), jnp.float32),
    )(*accs)
    return o


def kernel(x, adj, src, tgt, Msrc, Mtgt, Wf, bf, Ww, bw):
    src2d = src.astype(jnp.int32).reshape(E // CH, CH)
    tgt2d = tgt.astype(jnp.int32).reshape(E // CH, CH)
    zpad = jnp.zeros((DI, AW - 1), jnp.float32)
    W1a = jnp.concatenate([Wf[:, :DI].T, Ww[:, :DI].T, zpad], axis=1)
    W2a = jnp.concatenate([Wf[:, DI:].T, Ww[:, DI:].T, zpad], axis=1)
    bvec = jnp.concatenate(
        [bf, bw, jnp.zeros((AW - 1,), jnp.float32)]
    ).reshape(1, DZ)
    info = plsc.get_sparse_core_info()
    return _run(x, src2d, tgt2d, Mtgt, W1a, W2a, bvec,
                nc=info.num_cores, ns=info.num_subcores)


# speedup vs baseline: 1.1166x; 1.1166x over previous
import functools

import jax
import jax.numpy as jnp
from jax import lax
from jax.experimental import pallas as pl
from jax.experimental.pallas import tpu as pltpu
from jax.experimental.pallas import tpu_sc as plsc

N, E, DI, DO = 4096, 32768, 128, 128
EPS = 1e-06
EB = 1024
CH = 128
AW = 8
DZ = DO + AW
def _make_gather(nc, ns):
    nw = nc * ns
    epw = E // nw
    chw = epw // CH

    def body(x_hbm, src_hbm, tgt_hbm, xs_hbm, xt_hbm, sidx, tidx,
             ba0, ba1, bb0, bb1,
             sga0, sga1, sgb0, sgb1, swa0, swa1, swb0, swb1):
        bufs_a = (ba0, ba1)
        bufs_b = (bb0, bb1)
        sg_a = (sga0, sga1)
        sg_b = (sgb0, sgb1)
        sw_a = (swa0, swa1)
        sw_b = (swb0, swb1)
        wid = lax.axis_index("s") * nc + lax.axis_index("c")
        base = wid * epw
        pltpu.sync_copy(src_hbm.at[pl.ds(wid * chw, chw)], sidx)
        pltpu.sync_copy(tgt_hbm.at[pl.ds(wid * chw, chw)], tidx)
        prev = None
        wprev = [None, None]
        for k in range(chw):
            p = k & 1
            if wprev[p] is not None:
                wprev[p][0].wait()
                wprev[p][1].wait()
            ga = pltpu.async_copy(x_hbm.at[sidx.at[k]], bufs_a[p], sg_a[p])
            gb = pltpu.async_copy(x_hbm.at[tidx.at[k]], bufs_b[p], sg_b[p])
            if prev is not None:
                kp, pp, pga, pgb = prev
                pga.wait()
                pgb.wait()
                wa = pltpu.async_copy(
                    bufs_a[pp], xs_hbm.at[pl.ds(base + kp * CH, CH)], sw_a[pp])
                wb = pltpu.async_copy(
                    bufs_b[pp], xt_hbm.at[pl.ds(base + kp * CH, CH)], sw_b[pp])
                wprev[pp] = (wa, wb)
            prev = (k, p, ga, gb)
        kp, pp, pga, pgb = prev
        pga.wait()
        pgb.wait()
        wa = pltpu.async_copy(
            bufs_a[pp], xs_hbm.at[pl.ds(base + kp * CH, CH)], sw_a[pp])
        wb = pltpu.async_copy(
            bufs_b[pp], xt_hbm.at[pl.ds(base + kp * CH, CH)], sw_b[pp])
        wa.wait()
        wb.wait()
        other = pp ^ 1
        if wprev[other] is not None:
            wprev[other][0].wait()
            wprev[other][1].wait()

    return pl.kernel(
        body,
        out_type=[
            jax.ShapeDtypeStruct((E, DI), jnp.float32),
            jax.ShapeDtypeStruct((E, DI), jnp.float32),
        ],
        mesh=plsc.VectorSubcoreMesh(core_axis_name="c", subcore_axis_name="s"),
        scratch_types=[
            pltpu.VMEM((chw, CH), jnp.int32),
            pltpu.VMEM((chw, CH), jnp.int32),
            pltpu.VMEM((CH, DI), jnp.float32),
            pltpu.VMEM((CH, DI), jnp.float32),
            pltpu.VMEM((CH, DI), jnp.float32),
            pltpu.VMEM((CH, DI), jnp.float32),
            pltpu.SemaphoreType.DMA,
            pltpu.SemaphoreType.DMA,
            pltpu.SemaphoreType.DMA,
            pltpu.SemaphoreType.DMA,
            pltpu.SemaphoreType.DMA,
            pltpu.SemaphoreType.DMA,
            pltpu.SemaphoreType.DMA,
            pltpu.SemaphoreType.DMA,
        ],
    )


def _agg_body(mtgt_ref, xs_ref, xt_ref, w1_ref, w2_ref, b_ref, o_ref, acc_ref):
    i = pl.program_id(0)

    @pl.when(i == 0)
    def _init():
        acc_ref[...] = jnp.zeros_like(acc_ref)

    f = (jnp.dot(xs_ref[...], w1_ref[...], preferred_element_type=jnp.float32)
         + jnp.dot(xt_ref[...], w2_ref[...], preferred_element_type=jnp.float32)
         + b_ref[...])
    y = jnp.maximum(f[:, :DO], 0.0)
    a = f[:, DO:DO + 1]
    z = jnp.concatenate([y * a, f[:, DO:]], axis=1)
    acc_ref[...] += jnp.dot(mtgt_ref[...], z, preferred_element_type=jnp.float32)

    @pl.when(i == pl.num_programs(0) - 1)
    def _fini():
        o_ref[...] = acc_ref[:, :DO] / (acc_ref[:, DO:DO + 1] + EPS)


@functools.partial(jax.jit, static_argnames=("nc", "ns"))
def _run(x, src2d, tgt2d, Mtgt, W1a, W2a, bvec, nc, ns):
    xs, xt = _make_gather(nc, ns)(x, src2d, tgt2d)

    o = pl.pallas_call(
        _agg_body,
        grid=(E // EB,),
        in_specs=[
            pl.BlockSpec((N, EB), lambda i: (0, i)),
            pl.BlockSpec((EB, DI), lambda i: (i, 0)),
            pl.BlockSpec((EB, DI), lambda i: (i, 0)),
            pl.BlockSpec((DI, DZ), lambda i: (0, 0)),
            pl.BlockSpec((DI, DZ), lambda i: (0, 0)),
            pl.BlockSpec((1, DZ), lambda i: (0, 0)),
        ],
        out_specs=pl.BlockSpec((N, DO), lambda i: (0, 0)),
        out_shape=jax.ShapeDtypeStruct((N, DO), jnp.float32),
        scratch_shapes=[pltpu.VMEM((N, DZ), jnp.float32)],
    )(Mtgt, xs, xt, W1a, W2a, bvec)
    return o


def kernel(x, adj, src, tgt, Msrc, Mtgt, Wf, bf, Ww, bw):
    src2d = src.astype(jnp.int32).reshape(E // CH, CH)
    tgt2d = tgt.astype(jnp.int32).reshape(E // CH, CH)
    zpad = jnp.zeros((DI, AW - 1), jnp.float32)
    W1a = jnp.concatenate([Wf[:, :DI].T, Ww[:, :DI].T, zpad], axis=1)
    W2a = jnp.concatenate([Wf[:, DI:].T, Ww[:, DI:].T, zpad], axis=1)
    bvec = jnp.concatenate(
        [bf, bw, jnp.zeros((AW - 1,), jnp.float32)]
    ).reshape(1, DZ)
    info = plsc.get_sparse_core_info()
    return _run(x, src2d, tgt2d, Mtgt, W1a, W2a, bvec,
                nc=info.num_cores, ns=info.num_subcores)
